# baseline (speedup 1.0000x reference)
import functools

import jax
import jax.numpy as jnp
from jax.experimental import pallas as pl
from jax.experimental.pallas import tpu as pltpu

LANE = 128
BN_EPS = 1e-5
BLOCK_BYTES_TARGET = 20 * 1024 * 1024


def _round_up(a, m):
    return ((a + m - 1) // m) * m


def _pool4_kernel(x_ref, out_ref):
    s1 = jnp.sum(x_ref[...].astype(jnp.float32), axis=2)
    out_ref[0] = jnp.sum(s1, axis=2)


def _pool_core_body(x_ref, o_ref, buf, sem, osem, *, core, kpc, bblk, nbuf, hw):
    n_full = hw // LANE
    tail = hw % LANE
    hwpad = buf.shape[-1]
    C = buf.shape[2]
    base = core * kpc * bblk

    def _copy(b):
        dst = buf.at[b % nbuf]
        if hw != hwpad:
            dst = buf.at[b % nbuf, :, :, pl.ds(0, hw)]
        return pltpu.make_async_copy(
            x_ref.at[pl.ds(base + b * bblk, bblk)], dst, sem.at[b % nbuf])

    def _scoped(osc):
        for b in range(min(nbuf, kpc)):
            _copy(b).start()
        for b in range(kpc):
            _copy(b).wait()
            slot = buf.at[b % nbuf]
            acc = jnp.zeros((bblk, C, LANE), jnp.float32)
            for j in range(n_full):
                acc = acc + slot[:, :, j * LANE:(j + 1) * LANE].astype(jnp.float32)
            if tail:
                lane = jax.lax.broadcasted_iota(jnp.int32, (1, 1, LANE), 2)
                chunk = slot[:, :, n_full * LANE:(n_full + 1) * LANE]
                acc = acc + jnp.where(lane < tail, chunk.astype(jnp.float32), 0.0)
            if b + nbuf < kpc:
                _copy(b + nbuf).start()
            osc[b] = jnp.sum(acc, axis=2)
        out_cp = pltpu.make_async_copy(
            osc, o_ref.at[pl.ds(core * kpc, kpc)], osem)
        out_cp.start()
        out_cp.wait()

    pl.run_scoped(_scoped, pltpu.VMEM((kpc, bblk, C), jnp.float32))


def _head_kernel(psum_ref, wproj_ref, gamma_ref, beta_ref, wcls_ref,
                 gfeat_ref, feat_ref, cls_ref, *, inv_hw):
    pooled = psum_ref[...] * inv_hw
    gfeat = jnp.dot(pooled, wproj_ref[...],
                    preferred_element_type=jnp.float32)
    gfeat_ref[...] = gfeat
    mu = jnp.mean(gfeat, axis=0, keepdims=True)
    d = gfeat - mu
    var = jnp.mean(d * d, axis=0, keepdims=True)
    feat = d * jax.lax.rsqrt(var + BN_EPS) * gamma_ref[...] + beta_ref[...]
    feat_ref[...] = feat
    cls_ref[...] = jax.lax.dot_general(
        feat, wcls_ref[...], (((1,), (1,)), ((), ())),
        preferred_element_type=jnp.float32)


def kernel(x, wproj, gamma, beta, wcls):
    B, C, H, W = x.shape
    HW = H * W
    P = wproj.shape[1]
    NC = wcls.shape[0]
    hwpad = _round_up(HW, LANE)

    row_bytes = C * hwpad * jnp.dtype(x.dtype).itemsize
    nbuf = 4
    bblk = 1
    for cand in (8, 4, 2):
        if B % (2 * cand) == 0 and nbuf * cand * row_bytes <= 40 * 1024 * 1024:
            bblk = cand
            break
    ncores = 2 if B % (2 * bblk) == 0 else 1
    kpc = B // (ncores * bblk)
    nblocks = B // bblk

    vmem_limit = int(min(56 * 1024 * 1024,
                         nbuf * bblk * row_bytes + 4 * 1024 * 1024))

    psum = pl.pallas_call(
        _pool4_kernel,
        out_shape=jax.ShapeDtypeStruct((nblocks, bblk, C), jnp.float32),
        grid=(nblocks,),
        in_specs=[pl.BlockSpec((bblk, C, H, W), lambda k: (k, 0, 0, 0))],
        out_specs=pl.BlockSpec((1, bblk, C), lambda k: (k, 0, 0)),
        compiler_params=pltpu.CompilerParams(
            dimension_semantics=("arbitrary",),
            vmem_limit_bytes=vmem_limit,
        ),
    )(x).reshape(B, C)

    gfeat, feat, cls_score = pl.pallas_call(
        functools.partial(_head_kernel, inv_hw=1.0 / float(HW)),
        out_shape=(
            jax.ShapeDtypeStruct((B, P), jnp.float32),
            jax.ShapeDtypeStruct((B, P), jnp.float32),
            jax.ShapeDtypeStruct((B, NC), jnp.float32),
        ),
    )(psum, wproj.astype(jnp.float32), gamma.reshape(1, P).astype(jnp.float32),
      beta.reshape(1, P).astype(jnp.float32), wcls.astype(jnp.float32))

    return cls_score, gfeat, feat


# speedup vs baseline: 1.7675x; 1.7675x over previous
import functools

import jax
import jax.numpy as jnp
from jax.experimental import pallas as pl
from jax.experimental.pallas import tpu as pltpu

LANE = 128
BN_EPS = 1e-5
BLOCK_BYTES_TARGET = 20 * 1024 * 1024


def _round_up(a, m):
    return ((a + m - 1) // m) * m


def _pool_kernel(x_ref, out_ref, *, hw):
    n_full = hw // LANE
    tail = hw % LANE
    acc = jnp.zeros(x_ref.shape[:2] + (LANE,), jnp.float32)
    for j in range(n_full):
        acc = acc + x_ref[:, :, j * LANE:(j + 1) * LANE].astype(jnp.float32)
    if tail:
        lane = jax.lax.broadcasted_iota(jnp.int32, (1, 1, LANE), 2)
        chunk = x_ref[:, :, n_full * LANE:(n_full + 1) * LANE]
        acc = acc + jnp.where(lane < tail, chunk.astype(jnp.float32), 0.0)
    out_ref[0] = jnp.sum(acc, axis=2)


def _head_kernel(psum_ref, wproj_ref, gamma_ref, beta_ref, wcls_ref,
                 gfeat_ref, feat_ref, cls_ref, *, inv_hw):
    pooled = psum_ref[...] * inv_hw
    gfeat = jnp.dot(pooled, wproj_ref[...],
                    preferred_element_type=jnp.float32)
    gfeat_ref[...] = gfeat
    mu = jnp.mean(gfeat, axis=0, keepdims=True)
    d = gfeat - mu
    var = jnp.mean(d * d, axis=0, keepdims=True)
    feat = d * jax.lax.rsqrt(var + BN_EPS) * gamma_ref[...] + beta_ref[...]
    feat_ref[...] = feat
    cls_ref[...] = jax.lax.dot_general(
        feat, wcls_ref[...], (((1,), (1,)), ((), ())),
        preferred_element_type=jnp.float32)


def kernel(x, wproj, gamma, beta, wcls):
    B, C, H, W = x.shape
    HW = H * W
    P = wproj.shape[1]
    NC = wcls.shape[0]
    hwpad = _round_up(HW, LANE)

    row_bytes = C * hwpad * jnp.dtype(x.dtype).itemsize
    nbuf = 4
    bblk = 1
    for cand in (8, 4, 2):
        if B % (2 * cand) == 0 and nbuf * cand * row_bytes <= 40 * 1024 * 1024:
            bblk = cand
            break
    ncores = 2 if B % (2 * bblk) == 0 else 1
    kpc = B // (ncores * bblk)
    nblocks = B // bblk

    vmem_limit = int(min(56 * 1024 * 1024,
                         nbuf * bblk * row_bytes + 4 * 1024 * 1024))

    x3 = x.astype(jnp.bfloat16).reshape(B, C, HW)
    psum = pl.pallas_call(
        functools.partial(_pool_kernel, hw=HW),
        out_shape=jax.ShapeDtypeStruct((nblocks, bblk, C), jnp.float32),
        grid=(nblocks,),
        in_specs=[pl.BlockSpec((bblk, C, hwpad), lambda k: (k, 0, 0))],
        out_specs=pl.BlockSpec((1, bblk, C), lambda k: (k, 0, 0)),
        compiler_params=pltpu.CompilerParams(
            dimension_semantics=("arbitrary",),
            vmem_limit_bytes=vmem_limit,
        ),
    )(x3).reshape(B, C)

    gfeat, feat, cls_score = pl.pallas_call(
        functools.partial(_head_kernel, inv_hw=1.0 / float(HW)),
        out_shape=(
            jax.ShapeDtypeStruct((B, P), jnp.float32),
            jax.ShapeDtypeStruct((B, P), jnp.float32),
            jax.ShapeDtypeStruct((B, NC), jnp.float32),
        ),
    )(psum, wproj.astype(jnp.float32), gamma.reshape(1, P).astype(jnp.float32),
      beta.reshape(1, P).astype(jnp.float32), wcls.astype(jnp.float32))

    return cls_score, gfeat, feat


# speedup vs baseline: 1.8942x; 1.0717x over previous
import functools

import jax
import jax.numpy as jnp
from jax.experimental import pallas as pl
from jax.experimental.pallas import tpu as pltpu

LANE = 128
BN_EPS = 1e-5
BLOCK_BYTES_TARGET = 20 * 1024 * 1024


def _round_up(a, m):
    return ((a + m - 1) // m) * m


def _pool_kernel(x_ref, out_ref, *, hw):
    n_full = hw // LANE
    tail = hw % LANE
    acc = jnp.zeros(x_ref.shape[:2] + (LANE,), jnp.float32)
    for j in range(n_full):
        acc = acc + x_ref[:, :, j * LANE:(j + 1) * LANE].astype(jnp.float32)
    if tail:
        lane = jax.lax.broadcasted_iota(jnp.int32, (1, 1, LANE), 2)
        chunk = x_ref[:, :, n_full * LANE:(n_full + 1) * LANE]
        acc = acc + jnp.where(lane < tail, chunk.astype(jnp.float32), 0.0)
    out_ref[0] = jnp.sum(acc, axis=2)


def _head_kernel(psum_ref, wproj_ref, gamma_ref, beta_ref, wcls_ref,
                 gfeat_ref, feat_ref, cls_ref, *, inv_hw):
    pooled = psum_ref[...] * inv_hw
    gfeat = jnp.dot(pooled, wproj_ref[...],
                    preferred_element_type=jnp.float32)
    gfeat_ref[...] = gfeat
    mu = jnp.mean(gfeat, axis=0, keepdims=True)
    d = gfeat - mu
    var = jnp.mean(d * d, axis=0, keepdims=True)
    feat = d * jax.lax.rsqrt(var + BN_EPS) * gamma_ref[...] + beta_ref[...]
    feat_ref[...] = feat
    cls_ref[...] = jax.lax.dot_general(
        feat, wcls_ref[...], (((1,), (1,)), ((), ())),
        preferred_element_type=jnp.float32)


def kernel(x, wproj, gamma, beta, wcls):
    B, C, H, W = x.shape
    HW = H * W
    P = wproj.shape[1]
    NC = wcls.shape[0]
    hwpad = _round_up(HW, LANE)

    row_bytes = C * hwpad * jnp.dtype(x.dtype).itemsize
    nbuf = 4
    bblk = 1
    for cand in (8, 4, 2):
        if B % (2 * cand) == 0 and nbuf * cand * row_bytes <= 40 * 1024 * 1024:
            bblk = cand
            break
    ncores = 2 if B % (2 * bblk) == 0 else 1
    kpc = B // (ncores * bblk)
    nblocks = B // bblk

    vmem_limit = int(min(56 * 1024 * 1024,
                         nbuf * bblk * row_bytes + 4 * 1024 * 1024))

    x3 = x.reshape(B, C, HW)
    psum = pl.pallas_call(
        functools.partial(_pool_kernel, hw=HW),
        out_shape=jax.ShapeDtypeStruct((nblocks, bblk, C), jnp.float32),
        grid=(nblocks,),
        in_specs=[pl.BlockSpec((bblk, C, hwpad), lambda k: (k, 0, 0))],
        out_specs=pl.BlockSpec((1, bblk, C), lambda k: (k, 0, 0)),
        compiler_params=pltpu.CompilerParams(
            dimension_semantics=("arbitrary",),
            vmem_limit_bytes=vmem_limit,
        ),
    )(x3).reshape(B, C)

    gfeat, feat, cls_score = pl.pallas_call(
        functools.partial(_head_kernel, inv_hw=1.0 / float(HW)),
        out_shape=(
            jax.ShapeDtypeStruct((B, P), jnp.float32),
            jax.ShapeDtypeStruct((B, P), jnp.float32),
            jax.ShapeDtypeStruct((B, NC), jnp.float32),
        ),
    )(psum, wproj.astype(jnp.float32), gamma.reshape(1, P).astype(jnp.float32),
      beta.reshape(1, P).astype(jnp.float32), wcls.astype(jnp.float32))

    return cls_score, gfeat, feat


# speedup vs baseline: 1.9437x; 1.0261x over previous
import functools

import jax
import jax.numpy as jnp
from jax.experimental import pallas as pl
from jax.experimental.pallas import tpu as pltpu

LANE = 128
BN_EPS = 1e-5
BLOCK_BYTES_TARGET = 20 * 1024 * 1024


def _round_up(a, m):
    return ((a + m - 1) // m) * m


def _block_sums(x_ref, hw):
    n_full = hw // LANE
    tail = hw % LANE
    acc = jnp.zeros(x_ref.shape[:2] + (LANE,), jnp.float32)
    for j in range(n_full):
        acc = acc + x_ref[:, :, j * LANE:(j + 1) * LANE].astype(jnp.float32)
    if tail:
        lane = jax.lax.broadcasted_iota(jnp.int32, (1, 1, LANE), 2)
        chunk = x_ref[:, :, n_full * LANE:(n_full + 1) * LANE]
        acc = acc + jnp.where(lane < tail, chunk.astype(jnp.float32), 0.0)
    return jnp.sum(acc, axis=2)


def _fused_kernel(x_ref, wproj_h, gamma_h, beta_h, wcls_h,
                  gfeat_h, feat_h, cls_h,
                  psum, wp_v, ga_v, be_v, wc_v, gf_v, ft_v, cl_v, sem,
                  *, nblocks, bblk, hw, inv_hw):
    k = pl.program_id(0)

    @pl.when(k == 0)
    def _fetch_weights():
        pltpu.make_async_copy(wproj_h, wp_v, sem.at[0]).start()
        pltpu.make_async_copy(gamma_h, ga_v, sem.at[1]).start()
        pltpu.make_async_copy(beta_h, be_v, sem.at[2]).start()
        pltpu.make_async_copy(wcls_h, wc_v, sem.at[3]).start()

    psum[pl.ds(k * bblk, bblk)] = _block_sums(x_ref, hw)

    @pl.when(k == nblocks - 1)
    def _head():
        pltpu.make_async_copy(wproj_h, wp_v, sem.at[0]).wait()
        pltpu.make_async_copy(gamma_h, ga_v, sem.at[1]).wait()
        pltpu.make_async_copy(beta_h, be_v, sem.at[2]).wait()
        pltpu.make_async_copy(wcls_h, wc_v, sem.at[3]).wait()
        pooled = psum[...] * inv_hw
        gfeat = jnp.dot(pooled, wp_v[...],
                        preferred_element_type=jnp.float32)
        gf_v[...] = gfeat
        mu = jnp.mean(gfeat, axis=0, keepdims=True)
        d = gfeat - mu
        var = jnp.mean(d * d, axis=0, keepdims=True)
        feat = d * jax.lax.rsqrt(var + BN_EPS) * ga_v[...] + be_v[...]
        ft_v[...] = feat
        cl_v[...] = jax.lax.dot_general(
            feat, wc_v[...], (((1,), (1,)), ((), ())),
            preferred_element_type=jnp.float32)
        cp_g = pltpu.make_async_copy(gf_v, gfeat_h, sem.at[4])
        cp_f = pltpu.make_async_copy(ft_v, feat_h, sem.at[5])
        cp_c = pltpu.make_async_copy(cl_v, cls_h, sem.at[6])
        cp_g.start()
        cp_f.start()
        cp_c.start()
        cp_g.wait()
        cp_f.wait()
        cp_c.wait()


def _head_kernel(psum_ref, wproj_ref, gamma_ref, beta_ref, wcls_ref,
                 gfeat_ref, feat_ref, cls_ref, *, inv_hw):
    pooled = psum_ref[...] * inv_hw
    gfeat = jnp.dot(pooled, wproj_ref[...],
                    preferred_element_type=jnp.float32)
    gfeat_ref[...] = gfeat
    mu = jnp.mean(gfeat, axis=0, keepdims=True)
    d = gfeat - mu
    var = jnp.mean(d * d, axis=0, keepdims=True)
    feat = d * jax.lax.rsqrt(var + BN_EPS) * gamma_ref[...] + beta_ref[...]
    feat_ref[...] = feat
    cls_ref[...] = jax.lax.dot_general(
        feat, wcls_ref[...], (((1,), (1,)), ((), ())),
        preferred_element_type=jnp.float32)


def kernel(x, wproj, gamma, beta, wcls):
    B, C, H, W = x.shape
    HW = H * W
    P = wproj.shape[1]
    NC = wcls.shape[0]
    hwpad = _round_up(HW, LANE)

    row_bytes = C * hwpad * jnp.dtype(x.dtype).itemsize
    nbuf = 4
    bblk = 1
    for cand in (8, 4, 2):
        if B % (2 * cand) == 0 and nbuf * cand * row_bytes <= 40 * 1024 * 1024:
            bblk = cand
            break
    ncores = 2 if B % (2 * bblk) == 0 else 1
    kpc = B // (ncores * bblk)
    nblocks = B // bblk

    vmem_limit = int(min(56 * 1024 * 1024,
                         nbuf * bblk * row_bytes + 4 * 1024 * 1024))

    x3 = x.reshape(B, C, HW)
    hbm = pl.BlockSpec(memory_space=pltpu.MemorySpace.HBM)
    gfeat, feat, cls_score = pl.pallas_call(
        functools.partial(_fused_kernel, nblocks=nblocks, bblk=bblk,
                          hw=HW, inv_hw=1.0 / float(HW)),
        out_shape=(
            jax.ShapeDtypeStruct((B, P), jnp.float32),
            jax.ShapeDtypeStruct((B, P), jnp.float32),
            jax.ShapeDtypeStruct((B, NC), jnp.float32),
        ),
        grid=(nblocks,),
        in_specs=[pl.BlockSpec((bblk, C, hwpad), lambda k: (k, 0, 0)),
                  hbm, hbm, hbm, hbm],
        out_specs=(hbm, hbm, hbm),
        scratch_shapes=[
            pltpu.VMEM((B, C), jnp.float32),
            pltpu.VMEM((C, P), jnp.float32),
            pltpu.VMEM((1, P), jnp.float32),
            pltpu.VMEM((1, P), jnp.float32),
            pltpu.VMEM((NC, P), jnp.float32),
            pltpu.VMEM((B, P), jnp.float32),
            pltpu.VMEM((B, P), jnp.float32),
            pltpu.VMEM((B, NC), jnp.float32),
            pltpu.SemaphoreType.DMA((7,)),
        ],
        compiler_params=pltpu.CompilerParams(
            dimension_semantics=("arbitrary",),
            vmem_limit_bytes=vmem_limit,
        ),
    )(x3, wproj.astype(jnp.float32), gamma.reshape(1, P).astype(jnp.float32),
      beta.reshape(1, P).astype(jnp.float32), wcls.astype(jnp.float32))

    return cls_score, gfeat, feat


# speedup vs baseline: 1.9510x; 1.0038x over previous
import functools

import jax
import jax.numpy as jnp
from jax.experimental import pallas as pl
from jax.experimental.pallas import tpu as pltpu

LANE = 128
BN_EPS = 1e-5
BLOCK_BYTES_TARGET = 20 * 1024 * 1024


def _round_up(a, m):
    return ((a + m - 1) // m) * m


def _block_sums(x_ref, hw):
    n_full = hw // LANE
    tail = hw % LANE
    acc = jnp.zeros(x_ref.shape[:2] + (LANE,), jnp.float32)
    for j in range(n_full):
        acc = acc + x_ref[:, :, j * LANE:(j + 1) * LANE].astype(jnp.float32)
    if tail:
        lane = jax.lax.broadcasted_iota(jnp.int32, (1, 1, LANE), 2)
        chunk = x_ref[:, :, n_full * LANE:(n_full + 1) * LANE]
        acc = acc + jnp.where(lane < tail, chunk.astype(jnp.float32), 0.0)
    return jnp.sum(acc, axis=2)


def _fused_kernel(x_ref, wproj_h, gamma_h, beta_h, wcls_h,
                  gfeat_h, feat_h, cls_h,
                  psum, wp_v, ga_v, be_v, wc_v, gf_v, ft_v, cl_v, sem,
                  *, nblocks, bblk, hw, inv_hw):
    k = pl.program_id(0)

    @pl.when(k == 0)
    def _fetch_weights():
        pltpu.make_async_copy(wproj_h, wp_v, sem.at[0]).start()
        pltpu.make_async_copy(gamma_h, ga_v, sem.at[1]).start()
        pltpu.make_async_copy(beta_h, be_v, sem.at[2]).start()
        pltpu.make_async_copy(wcls_h, wc_v, sem.at[3]).start()

    psum[pl.ds(k * bblk, bblk)] = _block_sums(x_ref, hw)

    @pl.when(k == nblocks - 1)
    def _head():
        pltpu.make_async_copy(wproj_h, wp_v, sem.at[0]).wait()
        pltpu.make_async_copy(gamma_h, ga_v, sem.at[1]).wait()
        pltpu.make_async_copy(beta_h, be_v, sem.at[2]).wait()
        pltpu.make_async_copy(wcls_h, wc_v, sem.at[3]).wait()
        pooled = psum[...] * inv_hw
        gfeat = jnp.dot(pooled, wp_v[...],
                        preferred_element_type=jnp.float32)
        gf_v[...] = gfeat
        mu = jnp.mean(gfeat, axis=0, keepdims=True)
        d = gfeat - mu
        var = jnp.mean(d * d, axis=0, keepdims=True)
        feat = d * jax.lax.rsqrt(var + BN_EPS) * ga_v[...] + be_v[...]
        ft_v[...] = feat
        cl_v[...] = jax.lax.dot_general(
            feat, wc_v[...], (((1,), (1,)), ((), ())),
            preferred_element_type=jnp.float32)
        cp_g = pltpu.make_async_copy(gf_v, gfeat_h, sem.at[4])
        cp_f = pltpu.make_async_copy(ft_v, feat_h, sem.at[5])
        cp_c = pltpu.make_async_copy(cl_v, cls_h, sem.at[6])
        cp_g.start()
        cp_f.start()
        cp_c.start()
        cp_g.wait()
        cp_f.wait()
        cp_c.wait()


def _head_kernel(psum_ref, wproj_ref, gamma_ref, beta_ref, wcls_ref,
                 gfeat_ref, feat_ref, cls_ref, *, inv_hw):
    pooled = psum_ref[...] * inv_hw
    gfeat = jnp.dot(pooled, wproj_ref[...],
                    preferred_element_type=jnp.float32)
    gfeat_ref[...] = gfeat
    mu = jnp.mean(gfeat, axis=0, keepdims=True)
    d = gfeat - mu
    var = jnp.mean(d * d, axis=0, keepdims=True)
    feat = d * jax.lax.rsqrt(var + BN_EPS) * gamma_ref[...] + beta_ref[...]
    feat_ref[...] = feat
    cls_ref[...] = jax.lax.dot_general(
        feat, wcls_ref[...], (((1,), (1,)), ((), ())),
        preferred_element_type=jnp.float32)


def kernel(x, wproj, gamma, beta, wcls):
    B, C, H, W = x.shape
    HW = H * W
    P = wproj.shape[1]
    NC = wcls.shape[0]
    hwpad = _round_up(HW, LANE)

    row_bytes = C * hwpad * jnp.dtype(x.dtype).itemsize
    bblk = 1
    for cand in (8, 4, 2):
        if B % cand == 0 and 2 * cand * row_bytes <= 36 * 1024 * 1024:
            bblk = cand
            break
    nblocks = B // bblk

    vmem_limit = int(min(56 * 1024 * 1024,
                         2 * bblk * row_bytes + 6 * 1024 * 1024))

    x3 = x.reshape(B, C, HW)
    hbm = pl.BlockSpec(memory_space=pltpu.MemorySpace.HBM)
    gfeat, feat, cls_score = pl.pallas_call(
        functools.partial(_fused_kernel, nblocks=nblocks, bblk=bblk,
                          hw=HW, inv_hw=1.0 / float(HW)),
        out_shape=(
            jax.ShapeDtypeStruct((B, P), jnp.float32),
            jax.ShapeDtypeStruct((B, P), jnp.float32),
            jax.ShapeDtypeStruct((B, NC), jnp.float32),
        ),
        grid=(nblocks,),
        in_specs=[pl.BlockSpec((bblk, C, hwpad), lambda k: (k, 0, 0)),
                  hbm, hbm, hbm, hbm],
        out_specs=(hbm, hbm, hbm),
        scratch_shapes=[
            pltpu.VMEM((B, C), jnp.float32),
            pltpu.VMEM((C, P), jnp.float32),
            pltpu.VMEM((1, P), jnp.float32),
            pltpu.VMEM((1, P), jnp.float32),
            pltpu.VMEM((NC, P), jnp.float32),
            pltpu.VMEM((B, P), jnp.float32),
            pltpu.VMEM((B, P), jnp.float32),
            pltpu.VMEM((B, NC), jnp.float32),
            pltpu.SemaphoreType.DMA((7,)),
        ],
        compiler_params=pltpu.CompilerParams(
            dimension_semantics=("arbitrary",),
            vmem_limit_bytes=vmem_limit,
        ),
    )(x3, wproj.astype(jnp.float32), gamma.reshape(1, P).astype(jnp.float32),
      beta.reshape(1, P).astype(jnp.float32), wcls.astype(jnp.float32))

    return cls_score, gfeat, feat
